# baseline (speedup 1.0000x reference)
import functools

import jax
import jax.numpy as jnp
from jax import lax
from jax.experimental import pallas as pl
from jax.experimental.pallas import tpu as pltpu
from jax.experimental.pallas import tpu_sc as plsc

_NUM_CORES = 2
_NUM_SUBCORES = 16
_NUM_WORKERS = _NUM_CORES * _NUM_SUBCORES
_LANES = 16
_CHUNK = 16
_NBUF = 4
_DEPTH = 2


def _emb_lookup(ids_flat, word_embeddings, position_embeddings, *, batch,
                seqlen):
    _, d = word_embeddings.shape
    n = batch * seqlen
    s_per_w = seqlen // _NUM_WORKERS
    n_chunks = s_per_w // _CHUNK
    n_tasks = n_chunks * batch
    vecs_per_row = d // _LANES

    mesh = plsc.VectorSubcoreMesh(core_axis_name="c", subcore_axis_name="s")

    @functools.partial(
        pl.kernel,
        out_type=jax.ShapeDtypeStruct((n, d), jnp.float32),
        mesh=mesh,
        scratch_types=[
            pltpu.VMEM((batch, s_per_w), jnp.int32),
            pltpu.VMEM((_NBUF, _CHUNK, d), jnp.float32),
            pltpu.VMEM((2, _CHUNK, d), jnp.float32),
            [pltpu.SemaphoreType.DMA] * _NBUF,
            [pltpu.SemaphoreType.DMA] * _NBUF,
            [pltpu.SemaphoreType.DMA] * 2,
        ],
    )
    def body(ids_hbm, wtab_hbm, ptab_hbm, out_hbm, idx_v, rows_v, pos_v,
             sem_g, sem_o, sem_p):
        wid = lax.axis_index("s") * _NUM_CORES + lax.axis_index("c")
        s_base_w = wid * s_per_w

        idx_copies = [
            pltpu.make_async_copy(
                ids_hbm.at[pl.ds(b * seqlen + s_base_w, s_per_w)],
                idx_v.at[b],
                sem_p[0],
            )
            for b in range(batch)
        ]
        for cp in idx_copies:
            cp.start()
        for cp in idx_copies:
            cp.wait()

        def pos_copy(c, cpar):
            return pltpu.make_async_copy(
                ptab_hbm.at[pl.ds(s_base_w + c * _CHUNK, _CHUNK)],
                pos_v.at[cpar],
                sem_p[cpar],
            )

        def gather_copy(c, b, buf):
            return pltpu.make_async_copy(
                wtab_hbm.at[idx_v.at[b, pl.ds(c * _CHUNK, _CHUNK)]],
                rows_v.at[buf],
                sem_g[buf],
            )

        def out_copy(c, b, buf):
            return pltpu.make_async_copy(
                rows_v.at[buf],
                out_hbm.at[pl.ds(b * seqlen + s_base_w + c * _CHUNK, _CHUNK)],
                sem_o[buf],
            )

        pos_copy(0, 0).start()
        for t in range(_DEPTH):
            c0, b0 = divmod(t, batch)
            gather_copy(c0, b0, t % _NBUF).start()

        def run_chunk(c, cpar):
            pos_copy(c, cpar).wait()

            @pl.when(c + 1 < n_chunks)
            def _():
                pos_copy(c + 1, (cpar + 1) % 2).start()

            for b in range(batch):
                buf = (batch * cpar + b) % _NBUF

                bufn = (buf + _DEPTH) % _NBUF
                do, mo = divmod(b + _DEPTH - _NBUF, batch)
                bn = (b + _DEPTH) % batch
                dcn = (b + _DEPTH) // batch

                @pl.when(4 * c + b + _DEPTH - _NBUF >= 0)
                def _(do=do, mo=mo, bufn=bufn):
                    out_copy(c + do, mo, bufn).wait()

                @pl.when(4 * c + b + _DEPTH < n_tasks)
                def _(bn=bn, dcn=dcn, bufn=bufn):
                    gather_copy(c + dcn, bn, bufn).start()

                gather_copy(c, b, buf).wait()

                def add_rows(i2, _, buf=buf, cpar=cpar):
                    for r in range(2):
                        i = 2 * i2 + r
                        for j in range(vecs_per_row):
                            sl = pl.ds(j * _LANES, _LANES)
                            plsc.addupdate(rows_v.at[buf, i, sl],
                                           pos_v[cpar, i, sl])
                    return ()

                lax.fori_loop(0, _CHUNK // 2, add_rows, ())
                out_copy(c, b, buf).start()

        def chunk_pair(c2, _):
            run_chunk(2 * c2, 0)
            run_chunk(2 * c2 + 1, 1)
            return ()

        lax.fori_loop(0, n_chunks // 2, chunk_pair, ())

        for t in range(n_tasks - (_NBUF - _DEPTH), n_tasks):
            c0, b0 = divmod(t, batch)
            out_copy(c0, b0, t % _NBUF).wait()

    return body(ids_flat, word_embeddings, position_embeddings)


def kernel(input_ids, word_embeddings, position_embeddings):
    batch, seqlen = input_ids.shape
    _, d = word_embeddings.shape
    ids_flat = input_ids.reshape(batch * seqlen).astype(jnp.int32)
    out = _emb_lookup(
        ids_flat, word_embeddings, position_embeddings,
        batch=batch, seqlen=seqlen,
    )
    return out.reshape(batch, seqlen, d)


# speedup vs baseline: 1.5778x; 1.5778x over previous
import functools

import jax
import jax.numpy as jnp
from jax import lax
from jax.experimental import pallas as pl
from jax.experimental.pallas import tpu as pltpu
from jax.experimental.pallas import tpu_sc as plsc

_NUM_CORES = 2
_NUM_SUBCORES = 16
_NUM_WORKERS = _NUM_CORES * _NUM_SUBCORES
_LANES = 16
_CHUNK = 16
_NBUF = 4
_DEPTH = 2


def _emb_lookup(ids_flat, word_embeddings, position_embeddings, *, batch,
                seqlen):
    _, d = word_embeddings.shape
    n = batch * seqlen
    s_per_w = seqlen // _NUM_WORKERS
    n_chunks = s_per_w // _CHUNK
    n_tasks = n_chunks * batch
    vecs_per_row = d // _LANES

    mesh = plsc.VectorSubcoreMesh(core_axis_name="c", subcore_axis_name="s")

    @functools.partial(
        pl.kernel,
        out_type=jax.ShapeDtypeStruct((n, d), jnp.float32),
        mesh=mesh,
        scratch_types=[
            pltpu.VMEM((batch, s_per_w), jnp.int32),
            pltpu.VMEM((_NBUF, _CHUNK, d), jnp.float32),
            pltpu.VMEM((2, _CHUNK, d), jnp.float32),
            [pltpu.SemaphoreType.DMA] * _NBUF,
            [pltpu.SemaphoreType.DMA] * _NBUF,
            [pltpu.SemaphoreType.DMA] * 2,
        ],
    )
    def body(ids_hbm, wtab_hbm, ptab_hbm, out_hbm, idx_v, rows_v, pos_v,
             sem_g, sem_o, sem_p):
        wid = lax.axis_index("s") * _NUM_CORES + lax.axis_index("c")
        s_base_w = wid * s_per_w

        idx_copies = [
            pltpu.make_async_copy(
                ids_hbm.at[pl.ds(b * seqlen + s_base_w, s_per_w)],
                idx_v.at[b],
                sem_p[0],
            )
            for b in range(batch)
        ]
        for cp in idx_copies:
            cp.start()
        for cp in idx_copies:
            cp.wait()

        def pos_copy(c, cpar):
            return pltpu.make_async_copy(
                ptab_hbm.at[pl.ds(s_base_w + c * _CHUNK, _CHUNK)],
                pos_v.at[cpar],
                sem_p[cpar],
            )

        def gather_copy(c, b, buf):
            return pltpu.make_async_copy(
                wtab_hbm.at[idx_v.at[b, pl.ds(c * _CHUNK, _CHUNK)]],
                rows_v.at[buf],
                sem_g[buf],
            )

        def out_copy(c, b, buf):
            return pltpu.make_async_copy(
                rows_v.at[buf],
                out_hbm.at[pl.ds(b * seqlen + s_base_w + c * _CHUNK, _CHUNK)],
                sem_o[buf],
            )

        pos_copy(0, 0).start()
        for t in range(_DEPTH):
            c0, b0 = divmod(t, batch)
            gather_copy(c0, b0, t % _NBUF).start()

        def run_chunk(c, cpar):
            pos_copy(c, cpar).wait()

            @pl.when(c + 1 < n_chunks)
            def _():
                pos_copy(c + 1, (cpar + 1) % 2).start()

            for b in range(batch):
                buf = (batch * cpar + b) % _NBUF

                bufn = (buf + _DEPTH) % _NBUF
                do, mo = divmod(b + _DEPTH - _NBUF, batch)
                bn = (b + _DEPTH) % batch
                dcn = (b + _DEPTH) // batch

                @pl.when(4 * c + b + _DEPTH - _NBUF >= 0)
                def _(do=do, mo=mo, bufn=bufn):
                    out_copy(c + do, mo, bufn).wait()

                @pl.when(4 * c + b + _DEPTH < n_tasks)
                def _(bn=bn, dcn=dcn, bufn=bufn):
                    gather_copy(c + dcn, bn, bufn).start()

                gather_copy(c, b, buf).wait()

                def add_row(i, _, buf=buf, cpar=cpar):
                    for j in range(vecs_per_row):
                        sl = pl.ds(j * _LANES, _LANES)
                        plsc.addupdate(rows_v.at[buf, i, sl],
                                       pos_v[cpar, i, sl])
                    return ()

                lax.fori_loop(0, _CHUNK, add_row, ())
                out_copy(c, b, buf).start()

        def chunk_pair(c2, _):
            run_chunk(2 * c2, 0)
            run_chunk(2 * c2 + 1, 1)
            return ()

        lax.fori_loop(0, n_chunks // 2, chunk_pair, ())

        for t in range(n_tasks - (_NBUF - _DEPTH), n_tasks):
            c0, b0 = divmod(t, batch)
            out_copy(c0, b0, t % _NBUF).wait()

    return body(ids_flat, word_embeddings, position_embeddings)


def kernel(input_ids, word_embeddings, position_embeddings):
    batch, seqlen = input_ids.shape
    _, d = word_embeddings.shape
    ids_flat = input_ids.reshape(batch * seqlen).astype(jnp.int32)
    out = _emb_lookup(
        ids_flat, word_embeddings, position_embeddings,
        batch=batch, seqlen=seqlen,
    )
    return out.reshape(batch, seqlen, d)
